# baseline (speedup 1.0000x reference)
import functools

import jax
import jax.numpy as jnp
from jax import lax
from jax.experimental import pallas as pl
from jax.experimental.pallas import tpu as pltpu
from jax.experimental.pallas import tpu_sc as plsc

BATCH = 4096
SEQ_LEN = 200
EMBED_DIM = 64
PAIR_W = 2 * EMBED_DIM
CHUNK_A = 96
CHUNK_B = SEQ_LEN - CHUNK_A
NC = 2
NS = 16
NW = NC * NS
SPW = BATCH // NW
TPW = SPW * SEQ_LEN
LANES = 16


def _fire(emb2_hbm, pv_v, s, buf, sem):
  base = s * SEQ_LEN
  pltpu.async_copy(emb2_hbm.at[pv_v.at[pl.ds(base, CHUNK_A)]],
                   buf.at[pl.ds(0, CHUNK_A)], sem)
  pltpu.async_copy(emb2_hbm.at[pv_v.at[pl.ds(base + CHUNK_A, CHUNK_B)]],
                   buf.at[pl.ds(CHUNK_A, CHUNK_B)], sem)


def _drain(emb2_hbm, buf, sem):
  pltpu.make_async_copy(emb2_hbm.at[pl.ds(0, SEQ_LEN)], buf, sem).wait()


def _accum(buf, idx_v, acc_v, s):
  zero = jnp.zeros((LANES,), jnp.float32)
  tbase = s * SEQ_LEN

  def rows(carry, base, offs, n):
    a0, a1, a2, a3 = carry
    for k in range(n):
      row = base + k
      off = offs[k]
      a0 = a0 + buf[row, pl.ds(off + 0 * LANES, LANES)]
      a1 = a1 + buf[row, pl.ds(off + 1 * LANES, LANES)]
      a2 = a2 + buf[row, pl.ds(off + 2 * LANES, LANES)]
      a3 = a3 + buf[row, pl.ds(off + 3 * LANES, LANES)]
    return (a0, a1, a2, a3)

  def gbody(g, carry):
    offs = (idx_v[pl.ds(tbase + g * LANES, LANES)] & 1) * EMBED_DIM
    return rows(carry, g * LANES, offs, LANES)

  nfull = SEQ_LEN // LANES
  carry = lax.fori_loop(0, nfull, gbody, (zero, zero, zero, zero))
  toffs = (idx_v[pl.ds(tbase + nfull * LANES, LANES)] & 1) * EMBED_DIM
  a0, a1, a2, a3 = rows(carry, nfull * LANES, toffs, SEQ_LEN % LANES)
  obase = s * EMBED_DIM
  acc_v[pl.ds(obase + 0 * LANES, LANES)] = a0
  acc_v[pl.ds(obase + 1 * LANES, LANES)] = a1
  acc_v[pl.ds(obase + 2 * LANES, LANES)] = a2
  acc_v[pl.ds(obase + 3 * LANES, LANES)] = a3


@functools.partial(
    pl.kernel,
    mesh=plsc.VectorSubcoreMesh(core_axis_name="c", subcore_axis_name="s"),
    out_type=jax.ShapeDtypeStruct((BATCH * EMBED_DIM,), jnp.float32),
    scratch_types=[
        pltpu.VMEM((TPW + LANES,), jnp.int32),
        pltpu.VMEM((TPW,), jnp.int32),
        pltpu.VMEM((SEQ_LEN, PAIR_W), jnp.float32),
        pltpu.VMEM((SEQ_LEN, PAIR_W), jnp.float32),
        pltpu.VMEM((SPW * EMBED_DIM,), jnp.float32),
        pltpu.SemaphoreType.DMA,
        pltpu.SemaphoreType.DMA,
    ],
)
def _pool_sum(x_hbm, emb2_hbm, m_hbm, idx_v, pv_v, buf_a, buf_b, acc_v,
              sem_a, sem_b):
  wid = lax.axis_index("s") * NC + lax.axis_index("c")
  pltpu.sync_copy(x_hbm.at[pl.ds(wid * TPW, TPW)], idx_v.at[pl.ds(0, TPW)])

  def tbody(j, _):
    for k in range(4):
      o = (4 * j + k) * LANES
      pv_v[pl.ds(o, LANES)] = lax.shift_right_logical(
          idx_v[pl.ds(o, LANES)], 1)
    return 0

  lax.fori_loop(0, TPW // (4 * LANES), tbody, 0)

  _fire(emb2_hbm, pv_v, 0, buf_a, sem_a)

  def body(i, _):
    s = 2 * i

    @pl.when(s + 1 < SPW)
    def _():
      _fire(emb2_hbm, pv_v, s + 1, buf_b, sem_b)

    _drain(emb2_hbm, buf_a, sem_a)
    _accum(buf_a, idx_v, acc_v, s)

    @pl.when(s + 2 < SPW)
    def _():
      _fire(emb2_hbm, pv_v, s + 2, buf_a, sem_a)

    _drain(emb2_hbm, buf_b, sem_b)
    _accum(buf_b, idx_v, acc_v, s + 1)
    return 0

  lax.fori_loop(0, SPW // 2, body, 0)
  pltpu.sync_copy(acc_v, m_hbm.at[pl.ds(wid * SPW * EMBED_DIM,
                                        SPW * EMBED_DIM)])


def _mlp_body(m_ref, w1_ref, b1_ref, w2_ref, b2_ref, out_ref):
  m = m_ref[...] * (1.0 / SEQ_LEN)
  h = jnp.dot(m, w1_ref[...], preferred_element_type=jnp.float32)
  h = jnp.maximum(h + b1_ref[...], 0.0)
  out_ref[...] = (
      jnp.dot(h, w2_ref[...], preferred_element_type=jnp.float32)
      + b2_ref[...])


def _mlp(m_sum, W1, b1, W2p, b2p):
  blk = 512
  in_features = W1.shape[1]
  pad_cols = W2p.shape[1]
  return pl.pallas_call(
      _mlp_body,
      grid=(BATCH // blk,),
      in_specs=[
          pl.BlockSpec((blk, EMBED_DIM), lambda i: (i, 0)),
          pl.BlockSpec((EMBED_DIM, in_features), lambda i: (0, 0)),
          pl.BlockSpec((1, in_features), lambda i: (0, 0)),
          pl.BlockSpec((in_features, pad_cols), lambda i: (0, 0)),
          pl.BlockSpec((1, pad_cols), lambda i: (0, 0)),
      ],
      out_specs=pl.BlockSpec((blk, pad_cols), lambda i: (i, 0)),
      out_shape=jax.ShapeDtypeStruct((BATCH, pad_cols), jnp.float32),
  )(m_sum, W1, b1.reshape(1, -1), W2p, b2p.reshape(1, -1))


def kernel(x, emb, W1, b1, W2, b2):
  num_class = W2.shape[1]
  vocab = emb.shape[0]
  emb2 = emb.reshape(vocab // 2, PAIR_W)
  m_flat = _pool_sum(x.reshape(-1), emb2)
  m_sum = m_flat.reshape(BATCH, EMBED_DIM)
  pad_cols = 128
  W2p = jnp.pad(W2, ((0, 0), (0, pad_cols - num_class)))
  b2p = jnp.pad(b2, (0, pad_cols - num_class))
  out = _mlp(m_sum, W1, b1, W2p, b2p)
  return out[:, :num_class]
